# baseline (speedup 1.0000x reference)
import functools

import jax
import jax.numpy as jnp
from jax import lax
from jax.experimental import pallas as pl
from jax.experimental.pallas import tpu as pltpu
from jax.experimental.pallas import tpu_sc as plsc

_N, _E, _D, _ED = 9900, 316800, 128, 5
_NPG = 330
_BSZ = _N // _NPG
_NPAD = 9984
_HN = 3712
_NPH = 3
_RPH = _HN // 16
_DEN = 80
_C = 128
_G = _E // _C
_NW = 32
_BASE = _G // _NW
_EXTRA = _G - _BASE * _NW
_INV_SQRT = float(_D) ** -0.5


def _proj_body(x_ref, wq, bq, wk, bk, wv, bv, wsk, bsk,
               q_o, k_o, v_o, sk_o):
    xb = x_ref[...]
    q_o[...] = xb @ wq[...] + bq[...][None, :]
    k_o[...] = xb @ wk[...] + bk[...][None, :]
    v_o[...] = xb @ wv[...] + bv[...][None, :]
    sk_o[...] = xb @ wsk[...] + bsk[...][None, :]


def _prologue(xp, Wq, bq, Wk, bk, Wv, bv, Wskip, bskip):
    f32 = jnp.float32
    nblk = _NPAD // 128
    wspec = pl.BlockSpec((_D, _D), lambda i: (0, 0))
    bspec = pl.BlockSpec((_D,), lambda i: (0,))
    rspec = pl.BlockSpec((128, _D), lambda i: (i, 0))
    return pl.pallas_call(
        _proj_body,
        grid=(nblk,),
        in_specs=[rspec, wspec, bspec, wspec, bspec, wspec, bspec, wspec,
                  bspec],
        out_specs=[rspec, rspec, rspec, rspec],
        out_shape=[jax.ShapeDtypeStruct((_NPAD, _D), f32)] * 4,
    )(xp, Wq, bq, Wk, bk, Wv, bv, Wskip, bskip)


@functools.cache
def _get_edge_kernel():
  mesh = plsc.VectorSubcoreMesh(core_axis_name="c", subcore_axis_name="s")

  @functools.partial(
    pl.kernel,
    out_type=[
        jax.ShapeDtypeStruct((2 * _NPH, _HN, _D), jnp.float32),
        jax.ShapeDtypeStruct((2, _DEN, _D), jnp.float32),
        jax.ShapeDtypeStruct((_G, _C), jnp.float32),
    ],
    mesh=mesh,
    compiler_params=pltpu.CompilerParams(needs_layout_passes=False),
    scratch_types=[
        pltpu.VMEM((_C,), jnp.int32),
        pltpu.VMEM((_C,), jnp.int32),
        pltpu.VMEM((_C,), jnp.int32),
        pltpu.VMEM((_C,), jnp.int32),
        pltpu.VMEM((_C,), jnp.float32),
        pltpu.VMEM((_C, 16), jnp.float32),
        pltpu.VMEM((_C, 2 * _D), jnp.float32),
        pltpu.VMEM((_C, _D), jnp.float32),
        pltpu.VMEM((_C, _D), jnp.float32),
        pltpu.VMEM((_C, _D), jnp.float32),
        pltpu.VMEM((_ED, _D), jnp.float32),
        pltpu.VMEM((_D,), jnp.float32),
        pltpu.VMEM_SHARED((_HN, _D), jnp.float32),
        pltpu.VMEM_SHARED((_DEN, _D), jnp.float32),
        pltpu.SemaphoreType.DMA,
        pltpu.SemaphoreType.DMA,
    ],
  )
  def _edge_kernel(src_hbm, dst_hbm, ea_hbm, kv_hbm, q_hbm, v_hbm, we_hbm,
                   be_hbm, z128_hbm,
                   numv_out, den_out, ex_out,
                   src_v, dst_v, sidx_v, denrow_v, ex_v, ea_v, kv_rows,
                   q_rows, sv_rows, den_src, we_v, be_v, numv_sh, den_sh,
                   sem0, sem1):
    cid = lax.axis_index("c")
    sid = lax.axis_index("s")
    w = sid * 2 + cid
    lanes = lax.iota(jnp.int32, 16)
    f32 = jnp.float32

    pltpu.sync_copy(we_hbm, we_v)
    pltpu.sync_copy(be_hbm, be_v)
    ztile = sid * _RPH
    pltpu.sync_copy(z128_hbm.at[pl.ds(0, _C)], den_src)

    @pl.when(sid == jnp.int32(0))
    def _():
        pltpu.sync_copy(z128_hbm.at[pl.ds(0, _DEN)], den_sh)

    zero16i = jnp.zeros((16,), jnp.int32)

    def _fold_rows(dst_ref, both_halves):
        def _efold(e, carry):
            ea_row = ea_v[e, :]
            a0 = ea_row[0]
            a1 = ea_row[1]
            a2 = ea_row[2]
            a3 = ea_row[3]
            a4 = ea_row[4]
            for i in range(_D // 16):
                ev = (be_v[pl.ds(i * 16, 16)]
                      + a0 * we_v[0, pl.ds(i * 16, 16)]
                      + a1 * we_v[1, pl.ds(i * 16, 16)]
                      + a2 * we_v[2, pl.ds(i * 16, 16)]
                      + a3 * we_v[3, pl.ds(i * 16, 16)]
                      + a4 * we_v[4, pl.ds(i * 16, 16)])
                sl = pl.ds(i * 16, 16)
                dst_ref[e, sl] = dst_ref[e, sl] + ev
                if both_halves:
                    slv = pl.ds(_D + i * 16, 16)
                    dst_ref[e, slv] = dst_ref[e, slv] + ev
            return carry

        lax.fori_loop(0, _C, _efold, 0)

    def _scatter_phase(p, vrows_ref):
        for j in range(_C // 16):
            rows = lanes + jnp.int32(j * 16)
            dst16 = dst_v[pl.ds(j * 16, 16)]
            ex = ex_v[pl.ds(j * 16, 16)]
            r = dst16 - jnp.int32(p * _HN)
            inm = (r >= 0) & (r < _HN)
            exm = ex * inm.astype(f32)
            sidx_v[pl.ds(j * 16, 16)] = jnp.where(inm, r, 0)

            voff = vrows_ref.shape[1] - _D

            def _sstep2(d, carry):
                col = plsc.load_gather(vrows_ref,
                                       [rows, zero16i + (voff + d)])
                plsc.store_scatter(sv_rows, [rows, zero16i + d], col * exm)
                return carry

            lax.fori_loop(0, _D, _sstep2, 0, unroll=8)

        pltpu.sync_copy(sv_rows, numv_sh.at[sidx_v], add=True)

    def _chunk_p0(g):
        pltpu.sync_copy(src_hbm.at[g], src_v)
        pltpu.sync_copy(dst_hbm.at[g], dst_v)
        cp1 = pltpu.async_copy(kv_hbm.at[src_v], kv_rows, sem0)
        cp2 = pltpu.async_copy(q_hbm.at[dst_v], q_rows, sem1)
        pltpu.sync_copy(ea_hbm.at[g], ea_v)
        cp1.wait()
        cp2.wait()
        _fold_rows(kv_rows, True)

        for j in range(_C // 16):
            rows = lanes + jnp.int32(j * 16)
            dst16 = dst_v[pl.ds(j * 16, 16)]

            def _dstep(d, acc):
                cold = zero16i + d
                a = plsc.load_gather(kv_rows, [rows, cold])
                b = plsc.load_gather(q_rows, [rows, cold])
                return acc + a * b

            acc = lax.fori_loop(0, _D, _dstep, jnp.zeros((16,), f32),
                                unroll=8)
            ex = jnp.exp(acc * _INV_SQRT)
            ex_v[pl.ds(j * 16, 16)] = ex
            plsc.store_scatter(den_src, [rows, dst16 & 127], ex)
            denrow_v[pl.ds(j * 16, 16)] = dst16 >> 7

        pltpu.sync_copy(ex_v, ex_out.at[g])
        _scatter_phase(0, kv_rows)
        pltpu.sync_copy(den_src, den_sh.at[denrow_v], add=True)
        zf = jnp.zeros((16,), f32)
        for j in range(_C // 16):
            rows = lanes + jnp.int32(j * 16)
            dst16 = dst_v[pl.ds(j * 16, 16)]
            plsc.store_scatter(den_src, [rows, dst16 & 127], zf)

    def _chunk_pn(g, p):
        pltpu.sync_copy(src_hbm.at[g], src_v)
        pltpu.sync_copy(dst_hbm.at[g], dst_v)
        cp1 = pltpu.async_copy(v_hbm.at[src_v], q_rows, sem0)
        pltpu.sync_copy(ea_hbm.at[g], ea_v)
        pltpu.sync_copy(ex_out.at[g], ex_v)
        cp1.wait()
        _fold_rows(q_rows, False)
        _scatter_phase(p, q_rows)

    my_n = jnp.int32(_BASE) + (w < _EXTRA).astype(jnp.int32)

    for p in range(_NPH):
        pltpu.sync_copy(z128_hbm.at[pl.ds(0, _RPH)],
                        numv_sh.at[pl.ds(ztile, _RPH)])
        plsc.subcore_barrier()

        def _chunk_loop(i, carry):
            g = jnp.where(i < _BASE, w * _BASE + i,
                          jnp.int32(_NW * _BASE) + w)
            if p == 0:
                _chunk_p0(g)
            else:
                _chunk_pn(g, p)
            return carry

        lax.fori_loop(0, my_n, _chunk_loop, 0)

        plsc.subcore_barrier()
        pltpu.sync_copy(numv_sh.at[pl.ds(ztile, _RPH)],
                        numv_out.at[cid * _NPH + p, pl.ds(ztile, _RPH)])

    @pl.when(sid == jnp.int32(0))
    def _():
        pltpu.sync_copy(den_sh, den_out.at[cid])

  return _edge_kernel


def _epi_body(numv_ref, den_ref, skip_ref, gamma_ref, beta_ref,
              w1_ref, b1_ref, wr_ref, br_ref, s2_ref, out_ref):
    segs = [numv_ref[p] + numv_ref[_NPH + p] for p in range(_NPH)]
    nv = jnp.concatenate(segs, axis=0)[:_N]
    den = den_ref[0] + den_ref[1]
    out = nv / jnp.maximum(den, 1e-16)[:, None] + skip_ref[:_N, :]
    mu = jnp.mean(out, axis=0, keepdims=True)
    d0 = out - mu
    var = jnp.mean(d0 * d0, axis=0, keepdims=True)
    out = d0 * lax.rsqrt(var + 1e-5) * gamma_ref[...][None, :] \
        + beta_ref[...][None, :]
    out = jnp.maximum(out, 0.0)
    h = out.reshape(_N // 3, 3, _D)
    p1 = jnp.maximum(jnp.maximum(h[:, 0, :], h[:, 1, :]), h[:, 2, :])
    h2 = p1.reshape(_BSZ, 110, _D)[:, :108, :].reshape(_BSZ * 36, 3, _D)
    p2 = jnp.maximum(jnp.maximum(h2[:, 0, :], h2[:, 1, :]), h2[:, 2, :])
    h3 = p2.reshape(_BSZ * 18, 2, _D)
    p3 = jnp.maximum(h3[:, 0, :], h3[:, 1, :])
    h4 = jnp.maximum(p3 @ w1_ref[...] + b1_ref[...][None, :], 0.0)
    r = h4 @ wr_ref[...] + br_ref[...][None, :]
    c = s2_ref[...] @ r
    out_ref[...] = (1.0 / (1.0 + jnp.exp(-c)))[:, 0]


def _epilogue(numv_p, den2, skip, gamma, beta, W1, b1, Wr, br, s2):
    return pl.pallas_call(
        _epi_body,
        out_shape=jax.ShapeDtypeStruct((_BSZ,), jnp.float32),
    )(numv_p, den2, skip, gamma, beta, W1, b1, Wr, br, s2)


def kernel(x, edge_index, edge_attr, batch, Wq, bq, Wk, bk, Wv, bv, We, be,
           Wskip, bskip, gamma, beta, W1, b1, Wr, br):
    f32 = jnp.float32
    xp = jnp.pad(x, ((0, _NPAD - _N), (0, 0)))
    q, k, v, skip = _prologue(xp, Wq, bq, Wk, bk, Wv, bv, Wskip, bskip)
    kv = jnp.concatenate([k, v], axis=1)

    src = edge_index[0].reshape(_G, _C)
    dst = edge_index[1].reshape(_G, _C)
    ea3 = jnp.pad(edge_attr, ((0, 0), (0, 16 - _ED))).reshape(_G, _C, 16)
    z128 = jnp.zeros((_RPH, _D), f32)

    numv_p, den_p, _ = _get_edge_kernel()(src, dst, ea3, kv, q, v, We, be,
                                          z128)

    den2 = den_p.reshape(2, _DEN * _D)[:, :_N]

    s2 = jnp.kron(jnp.eye(_BSZ, dtype=f32),
                  jnp.full((1, 18), 1.0 / 18.0, f32))

    return _epilogue(numv_p, den2, skip, gamma, beta, W1, b1, Wr, br, s2)


# speedup vs baseline: 1.0085x; 1.0085x over previous
import functools

import jax
import jax.numpy as jnp
from jax import lax
from jax.experimental import pallas as pl
from jax.experimental.pallas import tpu as pltpu
from jax.experimental.pallas import tpu_sc as plsc

_N, _E, _D, _ED = 9900, 316800, 128, 5
_NPG = 330
_BSZ = _N // _NPG
_NPAD = 9984
_HN = 3712
_NPH = 3
_RPH = _HN // 16
_DEN = 80
_C = 128
_G = _E // _C
_NW = 32
_BASE = _G // _NW
_EXTRA = _G - _BASE * _NW
_INV_SQRT = float(_D) ** -0.5


def _proj_body(x_ref, wq, bq, wk, bk, wv, bv, wsk, bsk,
               q_o, k_o, v_o, sk_o):
    xb = x_ref[...]
    q_o[...] = xb @ wq[...] + bq[...][None, :]
    k_o[...] = xb @ wk[...] + bk[...][None, :]
    v_o[...] = xb @ wv[...] + bv[...][None, :]
    sk_o[...] = xb @ wsk[...] + bsk[...][None, :]


def _prologue(xp, Wq, bq, Wk, bk, Wv, bv, Wskip, bskip):
    f32 = jnp.float32
    nblk = _NPAD // 128
    wspec = pl.BlockSpec((_D, _D), lambda i: (0, 0))
    bspec = pl.BlockSpec((_D,), lambda i: (0,))
    rspec = pl.BlockSpec((128, _D), lambda i: (i, 0))
    return pl.pallas_call(
        _proj_body,
        grid=(nblk,),
        in_specs=[rspec, wspec, bspec, wspec, bspec, wspec, bspec, wspec,
                  bspec],
        out_specs=[rspec, rspec, rspec, rspec],
        out_shape=[jax.ShapeDtypeStruct((_NPAD, _D), f32)] * 4,
    )(xp, Wq, bq, Wk, bk, Wv, bv, Wskip, bskip)


@functools.cache
def _get_edge_kernel():
  mesh = plsc.VectorSubcoreMesh(core_axis_name="c", subcore_axis_name="s")

  @functools.partial(
    pl.kernel,
    out_type=[
        jax.ShapeDtypeStruct((2 * _NPH, _HN, _D), jnp.float32),
        jax.ShapeDtypeStruct((2, _DEN, _D), jnp.float32),
        jax.ShapeDtypeStruct((_G, _C), jnp.float32),
    ],
    mesh=mesh,
    compiler_params=pltpu.CompilerParams(needs_layout_passes=False),
    scratch_types=[
        pltpu.VMEM((2 * _C,), jnp.int32),
        pltpu.VMEM((_C,), jnp.int32),
        pltpu.VMEM((_C,), jnp.int32),
        pltpu.VMEM((_C,), jnp.float32),
        pltpu.VMEM((_C, 16), jnp.float32),
        pltpu.VMEM((_C, 2 * _D), jnp.float32),
        pltpu.VMEM((_C, _D), jnp.float32),
        pltpu.VMEM((_C, _D), jnp.float32),
        pltpu.VMEM((_C, _D), jnp.float32),
        pltpu.VMEM((_ED, _D), jnp.float32),
        pltpu.VMEM((_D,), jnp.float32),
        pltpu.VMEM_SHARED((_HN, _D), jnp.float32),
        pltpu.VMEM_SHARED((_DEN, _D), jnp.float32),
        pltpu.SemaphoreType.DMA,
        pltpu.SemaphoreType.DMA,
    ],
  )
  def _edge_kernel(sd_hbm, ea_hbm, kv_hbm, q_hbm, v_hbm, we_hbm,
                   be_hbm, z128_hbm,
                   numv_out, den_out, ex_out,
                   sd_v, sidx_v, denrow_v, ex_v, ea_v, kv_rows,
                   q_rows, sv_rows, den_src, we_v, be_v, numv_sh, den_sh,
                   sem0, sem1):
    src_v = sd_v.at[pl.ds(0, _C)]
    dst_v = sd_v.at[pl.ds(_C, _C)]
    cid = lax.axis_index("c")
    sid = lax.axis_index("s")
    w = sid * 2 + cid
    lanes = lax.iota(jnp.int32, 16)
    f32 = jnp.float32

    pltpu.sync_copy(we_hbm, we_v)
    pltpu.sync_copy(be_hbm, be_v)
    ztile = sid * _RPH
    pltpu.sync_copy(z128_hbm.at[pl.ds(0, _C)], den_src)

    @pl.when(sid == jnp.int32(0))
    def _():
        pltpu.sync_copy(z128_hbm.at[pl.ds(0, _DEN)], den_sh)

    zero16i = jnp.zeros((16,), jnp.int32)

    def _fold_rows(dst_ref, both_halves):
        def _efold(e, carry):
            ea_row = ea_v[e, :]
            a0 = ea_row[0]
            a1 = ea_row[1]
            a2 = ea_row[2]
            a3 = ea_row[3]
            a4 = ea_row[4]
            for i in range(_D // 16):
                ev = (be_v[pl.ds(i * 16, 16)]
                      + a0 * we_v[0, pl.ds(i * 16, 16)]
                      + a1 * we_v[1, pl.ds(i * 16, 16)]
                      + a2 * we_v[2, pl.ds(i * 16, 16)]
                      + a3 * we_v[3, pl.ds(i * 16, 16)]
                      + a4 * we_v[4, pl.ds(i * 16, 16)])
                sl = pl.ds(i * 16, 16)
                dst_ref[e, sl] = dst_ref[e, sl] + ev
                if both_halves:
                    slv = pl.ds(_D + i * 16, 16)
                    dst_ref[e, slv] = dst_ref[e, slv] + ev
            return carry

        lax.fori_loop(0, _C, _efold, 0)

    def _scatter_phase(p, vrows_ref):
        for j in range(_C // 16):
            rows = lanes + jnp.int32(j * 16)
            dst16 = dst_v[pl.ds(j * 16, 16)]
            ex = ex_v[pl.ds(j * 16, 16)]
            r = dst16 - jnp.int32(p * _HN)
            inm = (r >= 0) & (r < _HN)
            exm = ex * inm.astype(f32)
            sidx_v[pl.ds(j * 16, 16)] = jnp.where(inm, r, 0)

            voff = vrows_ref.shape[1] - _D

            def _sstep2(d, carry):
                col = plsc.load_gather(vrows_ref,
                                       [rows, zero16i + (voff + d)])
                plsc.store_scatter(sv_rows, [rows, zero16i + d], col * exm)
                return carry

            lax.fori_loop(0, _D, _sstep2, 0, unroll=8)

        pltpu.sync_copy(sv_rows, numv_sh.at[sidx_v], add=True)

    def _chunk_p0(g):
        pltpu.sync_copy(sd_hbm.at[g], sd_v)
        cp1 = pltpu.async_copy(kv_hbm.at[src_v], kv_rows, sem0)
        cp2 = pltpu.async_copy(q_hbm.at[dst_v], q_rows, sem1)
        pltpu.sync_copy(ea_hbm.at[g], ea_v)
        cp1.wait()
        cp2.wait()
        _fold_rows(kv_rows, True)

        for j in range(_C // 16):
            rows = lanes + jnp.int32(j * 16)
            dst16 = dst_v[pl.ds(j * 16, 16)]

            def _dstep(d, acc):
                cold = zero16i + d
                a = plsc.load_gather(kv_rows, [rows, cold])
                b = plsc.load_gather(q_rows, [rows, cold])
                return acc + a * b

            acc = lax.fori_loop(0, _D, _dstep, jnp.zeros((16,), f32),
                                unroll=8)
            ex = jnp.exp(acc * _INV_SQRT)
            ex_v[pl.ds(j * 16, 16)] = ex
            plsc.store_scatter(den_src, [rows, dst16 & 127], ex)
            denrow_v[pl.ds(j * 16, 16)] = dst16 >> 7

        pltpu.sync_copy(ex_v, ex_out.at[g])
        _scatter_phase(0, kv_rows)
        pltpu.sync_copy(den_src, den_sh.at[denrow_v], add=True)
        zf = jnp.zeros((16,), f32)
        for j in range(_C // 16):
            rows = lanes + jnp.int32(j * 16)
            dst16 = dst_v[pl.ds(j * 16, 16)]
            plsc.store_scatter(den_src, [rows, dst16 & 127], zf)

    def _chunk_pn(g, p):
        pltpu.sync_copy(sd_hbm.at[g], sd_v)
        cp1 = pltpu.async_copy(v_hbm.at[src_v], q_rows, sem0)
        pltpu.sync_copy(ea_hbm.at[g], ea_v)
        pltpu.sync_copy(ex_out.at[g], ex_v)
        cp1.wait()
        _fold_rows(q_rows, False)
        _scatter_phase(p, q_rows)

    my_n = jnp.int32(_BASE) + (w < _EXTRA).astype(jnp.int32)

    for p in range(_NPH):
        pltpu.sync_copy(z128_hbm.at[pl.ds(0, _RPH)],
                        numv_sh.at[pl.ds(ztile, _RPH)])
        plsc.subcore_barrier()

        def _chunk_loop(i, carry):
            g = jnp.where(i < _BASE, w * _BASE + i,
                          jnp.int32(_NW * _BASE) + w)
            if p == 0:
                _chunk_p0(g)
            else:
                _chunk_pn(g, p)
            return carry

        lax.fori_loop(0, my_n, _chunk_loop, 0)

        plsc.subcore_barrier()
        pltpu.sync_copy(numv_sh.at[pl.ds(ztile, _RPH)],
                        numv_out.at[cid * _NPH + p, pl.ds(ztile, _RPH)])

    @pl.when(sid == jnp.int32(0))
    def _():
        pltpu.sync_copy(den_sh, den_out.at[cid])

  return _edge_kernel


def _epi_body(numv_ref, den_ref, skip_ref, gamma_ref, beta_ref,
              w1_ref, b1_ref, wr_ref, br_ref, s2_ref, out_ref):
    segs = [numv_ref[p] + numv_ref[_NPH + p] for p in range(_NPH)]
    nv = jnp.concatenate(segs, axis=0)[:_N]
    den = den_ref[0] + den_ref[1]
    out = nv / jnp.maximum(den, 1e-16)[:, None] + skip_ref[:_N, :]
    mu = jnp.mean(out, axis=0, keepdims=True)
    d0 = out - mu
    var = jnp.mean(d0 * d0, axis=0, keepdims=True)
    out = d0 * lax.rsqrt(var + 1e-5) * gamma_ref[...][None, :] \
        + beta_ref[...][None, :]
    out = jnp.maximum(out, 0.0)
    h = out.reshape(_N // 3, 3, _D)
    p1 = jnp.maximum(jnp.maximum(h[:, 0, :], h[:, 1, :]), h[:, 2, :])
    h2 = p1.reshape(_BSZ, 110, _D)[:, :108, :].reshape(_BSZ * 36, 3, _D)
    p2 = jnp.maximum(jnp.maximum(h2[:, 0, :], h2[:, 1, :]), h2[:, 2, :])
    h3 = p2.reshape(_BSZ * 18, 2, _D)
    p3 = jnp.maximum(h3[:, 0, :], h3[:, 1, :])
    h4 = jnp.maximum(p3 @ w1_ref[...] + b1_ref[...][None, :], 0.0)
    r = h4 @ wr_ref[...] + br_ref[...][None, :]
    c = s2_ref[...] @ r
    out_ref[...] = (1.0 / (1.0 + jnp.exp(-c)))[:, 0]


def _epilogue(numv_p, den2, skip, gamma, beta, W1, b1, Wr, br, s2):
    return pl.pallas_call(
        _epi_body,
        out_shape=jax.ShapeDtypeStruct((_BSZ,), jnp.float32),
    )(numv_p, den2, skip, gamma, beta, W1, b1, Wr, br, s2)


def kernel(x, edge_index, edge_attr, batch, Wq, bq, Wk, bk, Wv, bv, We, be,
           Wskip, bskip, gamma, beta, W1, b1, Wr, br):
    f32 = jnp.float32
    xp = jnp.pad(x, ((0, _NPAD - _N), (0, 0)))
    q, k, v, skip = _prologue(xp, Wq, bq, Wk, bk, Wv, bv, Wskip, bskip)
    kv = jnp.concatenate([k, v], axis=1)

    sd = jnp.concatenate([edge_index[0].reshape(_G, _C),
                          edge_index[1].reshape(_G, _C)], axis=1)
    ea3 = jnp.pad(edge_attr, ((0, 0), (0, 16 - _ED))).reshape(_G, _C, 16)
    z128 = jnp.zeros((_RPH, _D), f32)

    numv_p, den_p, _ = _get_edge_kernel()(sd, ea3, kv, q, v, We, be, z128)

    den2 = den_p.reshape(2, _DEN * _D)[:, :_N]

    s2 = jnp.kron(jnp.eye(_BSZ, dtype=f32),
                  jnp.full((1, 18), 1.0 / 18.0, f32))

    return _epilogue(numv_p, den2, skip, gamma, beta, W1, b1, Wr, br, s2)
